# Initial kernel scaffold; baseline (speedup 1.0000x reference)
import jax
import jax.numpy as jnp
from jax.experimental import pallas as pl


def kernel(xyz1, points1, offset1, xyz2, points2, offset2, W1, b1, g1, be1, W2, b2, g2, be2):
    raise NotImplementedError("write your pallas kernel here")


# speedup vs baseline: 10.6728x; 10.6728x over previous
import functools

import jax
import jax.numpy as jnp
from jax import lax
from jax.experimental import pallas as pl
from jax.experimental.pallas import tpu as pltpu
from jax.experimental.pallas import tpu_sc as plsc

N1 = 16384
N2 = 4096
C1 = 128
C2 = 256
H1 = 256
H2 = 128

KNN_BLK = 256

NWORK = 32
QPW = N1 // NWORK
QC = 32
NCHUNK = QPW // QC


def _knn_body(q_ref, kt_ref, idx_ref, wspl_ref):
    q = q_ref[...]
    kt = kt_ref[...]
    qk = jnp.dot(q, kt, preferred_element_type=jnp.float32)
    q2 = jnp.sum(q * q, axis=1, keepdims=True)
    k2 = jnp.sum(kt * kt, axis=0, keepdims=True)
    d2 = q2 - 2.0 * qk + k2
    iota = lax.broadcasted_iota(jnp.int32, (KNN_BLK, N2), 1)
    big_i = jnp.int32(2**30)
    inf = jnp.float32(jnp.inf)
    vals = []
    idxs = []
    for r in range(3):
        m = jnp.min(d2, axis=1, keepdims=True)
        cand = jnp.where(d2 == m, iota, big_i)
        im = jnp.min(cand, axis=1, keepdims=True)
        vals.append(m)
        idxs.append(im)
        if r < 2:
            d2 = jnp.where(iota == im, inf, d2)
    dist = [jnp.sqrt(jnp.maximum(v, 0.0)) for v in vals]
    rec = [1.0 / (d + 1e-8) for d in dist]
    s = rec[0] + rec[1] + rec[2]
    w = [r_ / s for r_ in rec]
    idx_ref[...] = jnp.concatenate(idxs, axis=1)
    wspl_ref[...] = jnp.concatenate(
        [jnp.broadcast_to(wi, (KNN_BLK, 16)) for wi in w], axis=1)


def _knn(xyz1p, kt):
    grid = N1 // KNN_BLK
    return pl.pallas_call(
        _knn_body,
        grid=(grid,),
        in_specs=[
            pl.BlockSpec((KNN_BLK, 8), lambda i: (i, 0)),
            pl.BlockSpec((8, N2), lambda i: (0, 0)),
        ],
        out_specs=[
            pl.BlockSpec((KNN_BLK, 3), lambda i: (i, 0)),
            pl.BlockSpec((KNN_BLK, 48), lambda i: (i, 0)),
        ],
        out_shape=[
            jax.ShapeDtypeStruct((N1, 3), jnp.int32),
            jax.ShapeDtypeStruct((N1, 48), jnp.float32),
        ],
    )(xyz1p, kt)


def _sc_interp_body(idx_hbm, w_hbm, table_hbm, out_hbm,
                    idx_v, rows_v, w_v, out_v, sem):
    wid = lax.axis_index("s") * 2 + lax.axis_index("c")

    def chunk_body(ch, carry):
        qbase = wid * QPW + ch * QC
        rbase = qbase * 3
        pltpu.sync_copy(idx_hbm.at[pl.ds(rbase, QC * 3)], idx_v)
        pltpu.sync_copy(w_hbm.at[pl.ds(rbase * 16, QC * 48)], w_v)
        pltpu.async_copy(table_hbm.at[idx_v], rows_v, sem).wait()

        def q_body(q, carry2):
            w0 = w_v[pl.ds((3 * q + 0) * 16, 16)]
            w1 = w_v[pl.ds((3 * q + 1) * 16, 16)]
            w2 = w_v[pl.ds((3 * q + 2) * 16, 16)]
            for c in range(C2 // 16):
                sl = pl.ds(c * 16, 16)
                acc = (rows_v[3 * q + 0, sl] * w0
                       + rows_v[3 * q + 1, sl] * w1
                       + rows_v[3 * q + 2, sl] * w2)
                out_v[q, sl] = acc
            return carry2

        lax.fori_loop(0, QC, q_body, 0)
        pltpu.sync_copy(out_v, out_hbm.at[pl.ds(qbase, QC)])
        return carry

    lax.fori_loop(0, NCHUNK, chunk_body, 0)


def _sc_interp(idx_flat, w_flat, table):
    mesh = plsc.VectorSubcoreMesh(core_axis_name="c", subcore_axis_name="s")
    f = functools.partial(
        pl.kernel,
        out_type=jax.ShapeDtypeStruct((N1, C2), jnp.float32),
        mesh=mesh,
        scratch_types=[
            pltpu.VMEM((QC * 3,), jnp.int32),
            pltpu.VMEM((QC * 3, C2), jnp.float32),
            pltpu.VMEM((QC * 48,), jnp.float32),
            pltpu.VMEM((QC, C2), jnp.float32),
            pltpu.SemaphoreType.DMA,
        ],
    )(_sc_interp_body)
    return f(idx_flat, w_flat, table)


MLP_BLK = 1024


def _mlp1_body(p1_ref, it_ref, w1a_ref, w1b_ref, b1_ref, y1_ref, st1_ref):
    y = (jnp.dot(p1_ref[...], w1a_ref[...], preferred_element_type=jnp.float32)
         + jnp.dot(it_ref[...], w1b_ref[...], preferred_element_type=jnp.float32)
         + b1_ref[...])
    y1_ref[...] = y
    s = jnp.sum(y, axis=0, keepdims=True)
    ss = jnp.sum(y * y, axis=0, keepdims=True)
    st = jnp.concatenate([s, ss], axis=0)

    @pl.when(pl.program_id(0) == 0)
    def _():
        st1_ref[...] = st

    @pl.when(pl.program_id(0) != 0)
    def _():
        st1_ref[...] += st


def _mlp2_body(y1_ref, st1_ref, g1_ref, be1_ref, w2_ref, b2_ref,
               y2_ref, st2_ref):
    mu = st1_ref[0:1, :] * (1.0 / N1)
    var = st1_ref[1:2, :] * (1.0 / N1) - mu * mu
    a = g1_ref[...] / jnp.sqrt(var + 1e-5)
    c = be1_ref[...] - mu * a
    x = jnp.maximum(y1_ref[...] * a + c, 0.0)
    y = jnp.dot(x, w2_ref[...], preferred_element_type=jnp.float32) + b2_ref[...]
    y2_ref[...] = y
    s = jnp.sum(y, axis=0, keepdims=True)
    ss = jnp.sum(y * y, axis=0, keepdims=True)
    st = jnp.concatenate([s, ss], axis=0)

    @pl.when(pl.program_id(0) == 0)
    def _():
        st2_ref[...] = st

    @pl.when(pl.program_id(0) != 0)
    def _():
        st2_ref[...] += st


def _mlp3_body(y2_ref, st2_ref, g2_ref, be2_ref, out_ref):
    mu = st2_ref[0:1, :] * (1.0 / N1)
    var = st2_ref[1:2, :] * (1.0 / N1) - mu * mu
    a = g2_ref[...] / jnp.sqrt(var + 1e-5)
    c = be2_ref[...] - mu * a
    out_ref[...] = jnp.maximum(y2_ref[...] * a + c, 0.0)


def _mlp(points1, interp, W1, b1, g1, be1, W2, b2, g2, be2):
    grid = N1 // MLP_BLK
    w1a = W1[:C1]
    w1b = W1[C1:]
    y1, st1 = pl.pallas_call(
        _mlp1_body,
        grid=(grid,),
        in_specs=[
            pl.BlockSpec((MLP_BLK, C1), lambda i: (i, 0)),
            pl.BlockSpec((MLP_BLK, C2), lambda i: (i, 0)),
            pl.BlockSpec((C1, H1), lambda i: (0, 0)),
            pl.BlockSpec((C2, H1), lambda i: (0, 0)),
            pl.BlockSpec((1, H1), lambda i: (0, 0)),
        ],
        out_specs=[
            pl.BlockSpec((MLP_BLK, H1), lambda i: (i, 0)),
            pl.BlockSpec((2, H1), lambda i: (0, 0)),
        ],
        out_shape=[
            jax.ShapeDtypeStruct((N1, H1), jnp.float32),
            jax.ShapeDtypeStruct((2, H1), jnp.float32),
        ],
    )(points1, interp, w1a, w1b, b1.reshape(1, H1))

    y2, st2 = pl.pallas_call(
        _mlp2_body,
        grid=(grid,),
        in_specs=[
            pl.BlockSpec((MLP_BLK, H1), lambda i: (i, 0)),
            pl.BlockSpec((2, H1), lambda i: (0, 0)),
            pl.BlockSpec((1, H1), lambda i: (0, 0)),
            pl.BlockSpec((1, H1), lambda i: (0, 0)),
            pl.BlockSpec((H1, H2), lambda i: (0, 0)),
            pl.BlockSpec((1, H2), lambda i: (0, 0)),
        ],
        out_specs=[
            pl.BlockSpec((MLP_BLK, H2), lambda i: (i, 0)),
            pl.BlockSpec((2, H2), lambda i: (0, 0)),
        ],
        out_shape=[
            jax.ShapeDtypeStruct((N1, H2), jnp.float32),
            jax.ShapeDtypeStruct((2, H2), jnp.float32),
        ],
    )(y1, st1, g1.reshape(1, H1), be1.reshape(1, H1), W2, b2.reshape(1, H2))

    out = pl.pallas_call(
        _mlp3_body,
        grid=(grid,),
        in_specs=[
            pl.BlockSpec((MLP_BLK, H2), lambda i: (i, 0)),
            pl.BlockSpec((2, H2), lambda i: (0, 0)),
            pl.BlockSpec((1, H2), lambda i: (0, 0)),
            pl.BlockSpec((1, H2), lambda i: (0, 0)),
        ],
        out_specs=pl.BlockSpec((MLP_BLK, H2), lambda i: (i, 0)),
        out_shape=jax.ShapeDtypeStruct((N1, H2), jnp.float32),
    )(y2, st2, g2.reshape(1, H2), be2.reshape(1, H2))
    return out


def kernel(xyz1, points1, offset1, xyz2, points2, offset2,
           W1, b1, g1, be1, W2, b2, g2, be2):
    del offset1, offset2
    xyz1p = jnp.concatenate(
        [xyz1, jnp.zeros((N1, 5), jnp.float32)], axis=1)
    kt = jnp.concatenate(
        [xyz2.T, jnp.zeros((5, N2), jnp.float32)], axis=0)
    idx, wspl = _knn(xyz1p, kt)
    interp = _sc_interp(idx.reshape(N1 * 3), wspl.reshape(N1 * 48), points2)
    return _mlp(points1, interp, W1, b1, g1, be1, W2, b2, g2, be2)


# speedup vs baseline: 13.1588x; 1.2329x over previous
import functools

import jax
import jax.numpy as jnp
from jax import lax
from jax.experimental import pallas as pl
from jax.experimental.pallas import tpu as pltpu
from jax.experimental.pallas import tpu_sc as plsc

N1 = 16384
N2 = 4096
C1 = 128
C2 = 256
H1 = 256
H2 = 128

KNN_BLK = 512

NWORK = 32
QPW = N1 // NWORK
QC = 64
NCHUNK = QPW // QC


def _knn_body(q_ref, kt_ref, idx_ref, wspl_ref):
    q = q_ref[...]
    kt = kt_ref[...]
    qk = jnp.dot(q, kt, preferred_element_type=jnp.float32)
    q2 = jnp.sum(q * q, axis=1, keepdims=True)
    k2 = jnp.sum(kt * kt, axis=0, keepdims=True)
    d2 = q2 - 2.0 * qk + k2
    iota = lax.broadcasted_iota(jnp.int32, (KNN_BLK, N2), 1).astype(jnp.float32)
    big = jnp.float32(3e38)
    inf = jnp.float32(jnp.inf)
    vals = []
    idxs = []
    for r in range(3):
        m = jnp.min(d2, axis=1, keepdims=True)
        cand = jnp.where(d2 == m, iota, big)
        im = jnp.min(cand, axis=1, keepdims=True)
        vals.append(m)
        idxs.append(im)
        if r < 2:
            d2 = jnp.where(iota == im, inf, d2)
    dist = [jnp.sqrt(jnp.maximum(v, 0.0)) for v in vals]
    rec = [1.0 / (d + 1e-8) for d in dist]
    s = rec[0] + rec[1] + rec[2]
    w = [r_ / s for r_ in rec]
    idx_ref[...] = jnp.concatenate(
        [i.astype(jnp.int32) for i in idxs], axis=1)
    wspl_ref[...] = jnp.concatenate(
        [jnp.broadcast_to(wi, (KNN_BLK, 16)) for wi in w], axis=1)


def _knn(xyz1p, kt):
    grid = N1 // KNN_BLK
    return pl.pallas_call(
        _knn_body,
        grid=(grid,),
        in_specs=[
            pl.BlockSpec((KNN_BLK, 8), lambda i: (i, 0)),
            pl.BlockSpec((8, N2), lambda i: (0, 0)),
        ],
        out_specs=[
            pl.BlockSpec((KNN_BLK, 3), lambda i: (i, 0)),
            pl.BlockSpec((KNN_BLK, 48), lambda i: (i, 0)),
        ],
        out_shape=[
            jax.ShapeDtypeStruct((N1, 3), jnp.int32),
            jax.ShapeDtypeStruct((N1, 48), jnp.float32),
        ],
    )(xyz1p, kt)


def _sc_interp_body(idx_hbm, w_hbm, table_hbm, out_hbm,
                    idx_v, rows_a, rows_b, w_a, w_b, out_v, sem, wsem):
    wid = lax.axis_index("s") * 2 + lax.axis_index("c")
    qw = wid * QPW
    pltpu.sync_copy(idx_hbm.at[pl.ds(qw * 3, QPW * 3)], idx_v)

    rows = [rows_a, rows_b]
    ws = [w_a, w_b]

    def fire(ch, buf):
        base = ch * QC * 3
        pltpu.async_copy(table_hbm.at[idx_v.at[pl.ds(base, 96)]],
                         rows[buf].at[pl.ds(0, 96)], sem)
        pltpu.async_copy(table_hbm.at[idx_v.at[pl.ds(base + 96, 96)]],
                         rows[buf].at[pl.ds(96, 96)], sem)
        pltpu.async_copy(w_hbm.at[pl.ds((qw * 3 + base) * 16, QC * 48)],
                         ws[buf], wsem)

    def drain(buf):
        pltpu.make_async_copy(table_hbm.at[idx_v.at[pl.ds(0, 96)]],
                              rows[buf].at[pl.ds(0, 96)], sem).wait()
        pltpu.make_async_copy(table_hbm.at[idx_v.at[pl.ds(0, 96)]],
                              rows[buf].at[pl.ds(96, 96)], sem).wait()
        pltpu.make_async_copy(w_hbm.at[pl.ds(0, QC * 48)], ws[buf],
                              wsem).wait()

    fire(0, 0)
    for ch in range(NCHUNK):
        buf = ch % 2
        if ch + 1 < NCHUNK:
            fire(ch + 1, 1 - buf)
        drain(buf)
        rows_v = rows[buf]
        w_v = ws[buf]

        def q_body(q, carry2):
            w0 = w_v[pl.ds((3 * q + 0) * 16, 16)]
            w1 = w_v[pl.ds((3 * q + 1) * 16, 16)]
            w2 = w_v[pl.ds((3 * q + 2) * 16, 16)]
            for c in range(C2 // 16):
                sl = pl.ds(c * 16, 16)
                acc = (rows_v[3 * q + 0, sl] * w0
                       + rows_v[3 * q + 1, sl] * w1
                       + rows_v[3 * q + 2, sl] * w2)
                out_v[q, sl] = acc
            return carry2

        lax.fori_loop(0, QC, q_body, 0)
        pltpu.sync_copy(out_v, out_hbm.at[pl.ds(qw + ch * QC, QC)])


def _sc_interp(idx_flat, w_flat, table):
    mesh = plsc.VectorSubcoreMesh(core_axis_name="c", subcore_axis_name="s")
    f = functools.partial(
        pl.kernel,
        out_type=jax.ShapeDtypeStruct((N1, C2), jnp.float32),
        mesh=mesh,
        scratch_types=[
            pltpu.VMEM((QPW * 3,), jnp.int32),
            pltpu.VMEM((QC * 3, C2), jnp.float32),
            pltpu.VMEM((QC * 3, C2), jnp.float32),
            pltpu.VMEM((QC * 48,), jnp.float32),
            pltpu.VMEM((QC * 48,), jnp.float32),
            pltpu.VMEM((QC, C2), jnp.float32),
            pltpu.SemaphoreType.DMA,
            pltpu.SemaphoreType.DMA,
        ],
    )(_sc_interp_body)
    return f(idx_flat, w_flat, table)


MLP_BLK = 1024


def _mlp1_body(p1_ref, it_ref, w1a_ref, w1b_ref, b1_ref, y1_ref, st1_ref):
    y = (jnp.dot(p1_ref[...], w1a_ref[...], preferred_element_type=jnp.float32)
         + jnp.dot(it_ref[...], w1b_ref[...], preferred_element_type=jnp.float32)
         + b1_ref[...])
    y1_ref[...] = y
    s = jnp.sum(y, axis=0, keepdims=True)
    ss = jnp.sum(y * y, axis=0, keepdims=True)
    st = jnp.concatenate([s, ss], axis=0)

    @pl.when(pl.program_id(0) == 0)
    def _():
        st1_ref[...] = st

    @pl.when(pl.program_id(0) != 0)
    def _():
        st1_ref[...] += st


def _mlp2_body(y1_ref, st1_ref, g1_ref, be1_ref, w2_ref, b2_ref,
               y2_ref, st2_ref):
    mu = st1_ref[0:1, :] * (1.0 / N1)
    var = st1_ref[1:2, :] * (1.0 / N1) - mu * mu
    a = g1_ref[...] / jnp.sqrt(var + 1e-5)
    c = be1_ref[...] - mu * a
    x = jnp.maximum(y1_ref[...] * a + c, 0.0)
    y = jnp.dot(x, w2_ref[...], preferred_element_type=jnp.float32) + b2_ref[...]
    y2_ref[...] = y
    s = jnp.sum(y, axis=0, keepdims=True)
    ss = jnp.sum(y * y, axis=0, keepdims=True)
    st = jnp.concatenate([s, ss], axis=0)

    @pl.when(pl.program_id(0) == 0)
    def _():
        st2_ref[...] = st

    @pl.when(pl.program_id(0) != 0)
    def _():
        st2_ref[...] += st


def _mlp3_body(y2_ref, st2_ref, g2_ref, be2_ref, out_ref):
    mu = st2_ref[0:1, :] * (1.0 / N1)
    var = st2_ref[1:2, :] * (1.0 / N1) - mu * mu
    a = g2_ref[...] / jnp.sqrt(var + 1e-5)
    c = be2_ref[...] - mu * a
    out_ref[...] = jnp.maximum(y2_ref[...] * a + c, 0.0)


def _mlp(points1, interp, W1, b1, g1, be1, W2, b2, g2, be2):
    grid = N1 // MLP_BLK
    w1a = W1[:C1]
    w1b = W1[C1:]
    y1, st1 = pl.pallas_call(
        _mlp1_body,
        grid=(grid,),
        in_specs=[
            pl.BlockSpec((MLP_BLK, C1), lambda i: (i, 0)),
            pl.BlockSpec((MLP_BLK, C2), lambda i: (i, 0)),
            pl.BlockSpec((C1, H1), lambda i: (0, 0)),
            pl.BlockSpec((C2, H1), lambda i: (0, 0)),
            pl.BlockSpec((1, H1), lambda i: (0, 0)),
        ],
        out_specs=[
            pl.BlockSpec((MLP_BLK, H1), lambda i: (i, 0)),
            pl.BlockSpec((2, H1), lambda i: (0, 0)),
        ],
        out_shape=[
            jax.ShapeDtypeStruct((N1, H1), jnp.float32),
            jax.ShapeDtypeStruct((2, H1), jnp.float32),
        ],
    )(points1, interp, w1a, w1b, b1.reshape(1, H1))

    y2, st2 = pl.pallas_call(
        _mlp2_body,
        grid=(grid,),
        in_specs=[
            pl.BlockSpec((MLP_BLK, H1), lambda i: (i, 0)),
            pl.BlockSpec((2, H1), lambda i: (0, 0)),
            pl.BlockSpec((1, H1), lambda i: (0, 0)),
            pl.BlockSpec((1, H1), lambda i: (0, 0)),
            pl.BlockSpec((H1, H2), lambda i: (0, 0)),
            pl.BlockSpec((1, H2), lambda i: (0, 0)),
        ],
        out_specs=[
            pl.BlockSpec((MLP_BLK, H2), lambda i: (i, 0)),
            pl.BlockSpec((2, H2), lambda i: (0, 0)),
        ],
        out_shape=[
            jax.ShapeDtypeStruct((N1, H2), jnp.float32),
            jax.ShapeDtypeStruct((2, H2), jnp.float32),
        ],
    )(y1, st1, g1.reshape(1, H1), be1.reshape(1, H1), W2, b2.reshape(1, H2))

    out = pl.pallas_call(
        _mlp3_body,
        grid=(grid,),
        in_specs=[
            pl.BlockSpec((MLP_BLK, H2), lambda i: (i, 0)),
            pl.BlockSpec((2, H2), lambda i: (0, 0)),
            pl.BlockSpec((1, H2), lambda i: (0, 0)),
            pl.BlockSpec((1, H2), lambda i: (0, 0)),
        ],
        out_specs=pl.BlockSpec((MLP_BLK, H2), lambda i: (i, 0)),
        out_shape=jax.ShapeDtypeStruct((N1, H2), jnp.float32),
    )(y2, st2, g2.reshape(1, H2), be2.reshape(1, H2))
    return out


def kernel(xyz1, points1, offset1, xyz2, points2, offset2,
           W1, b1, g1, be1, W2, b2, g2, be2):
    del offset1, offset2
    xyz1p = jnp.concatenate(
        [xyz1, jnp.zeros((N1, 5), jnp.float32)], axis=1)
    kt = jnp.concatenate(
        [xyz2.T, jnp.zeros((5, N2), jnp.float32)], axis=0)
    idx, wspl = _knn(xyz1p, kt)
    interp = _sc_interp(idx.reshape(N1 * 3), wspl.reshape(N1 * 48), points2)
    return _mlp(points1, interp, W1, b1, g1, be1, W2, b2, g2, be2)


# speedup vs baseline: 13.8775x; 1.0546x over previous
import functools

import jax
import jax.numpy as jnp
from jax import lax
from jax.experimental import pallas as pl
from jax.experimental.pallas import tpu as pltpu
from jax.experimental.pallas import tpu_sc as plsc

N1 = 16384
N2 = 4096
C1 = 128
C2 = 256
H1 = 256
H2 = 128

KNN_BLK = 512

NWORK = 32
QPW = N1 // NWORK
QC = 64
NCHUNK = QPW // QC


def _knn_body(q_ref, kt_ref, idx_ref, wspl_ref):
    q = q_ref[...]
    kt = kt_ref[...]
    qk = jnp.dot(q, kt, preferred_element_type=jnp.float32)
    q2 = jnp.sum(q * q, axis=1, keepdims=True)
    k2 = jnp.sum(kt * kt, axis=0, keepdims=True)
    d2 = q2 - 2.0 * qk + k2
    iota = lax.broadcasted_iota(jnp.int32, (KNN_BLK, N2), 1).astype(jnp.float32)
    big = jnp.float32(3e38)
    inf = jnp.float32(jnp.inf)
    vals = []
    idxs = []
    for r in range(3):
        m = jnp.min(d2, axis=1, keepdims=True)
        cand = jnp.where(d2 == m, iota, big)
        im = jnp.min(cand, axis=1, keepdims=True)
        vals.append(m)
        idxs.append(im)
        if r < 2:
            d2 = jnp.where(iota == im, inf, d2)
    dist = [jnp.sqrt(jnp.maximum(v, 0.0)) for v in vals]
    rec = [1.0 / (d + 1e-8) for d in dist]
    s = rec[0] + rec[1] + rec[2]
    w = [r_ / s for r_ in rec]
    idx_ref[...] = jnp.concatenate(
        [i.astype(jnp.int32) for i in idxs], axis=1)
    wspl_ref[...] = jnp.concatenate(
        [jnp.broadcast_to(wi, (KNN_BLK, 16)) for wi in w], axis=1)


def _knn(xyz1p, kt):
    grid = N1 // KNN_BLK
    return pl.pallas_call(
        _knn_body,
        grid=(grid,),
        in_specs=[
            pl.BlockSpec((KNN_BLK, 8), lambda i: (i, 0)),
            pl.BlockSpec((8, N2), lambda i: (0, 0)),
        ],
        out_specs=[
            pl.BlockSpec((KNN_BLK, 3), lambda i: (i, 0)),
            pl.BlockSpec((KNN_BLK, 48), lambda i: (i, 0)),
        ],
        out_shape=[
            jax.ShapeDtypeStruct((N1, 3), jnp.int32),
            jax.ShapeDtypeStruct((N1, 48), jnp.float32),
        ],
    )(xyz1p, kt)


def _sc_interp_body(idx_hbm, w_hbm, table_hbm, out_hbm,
                    idx_v, rows_a, rows_b, w_a, w_b, out_v, sem, wsem):
    wid = lax.axis_index("s") * 2 + lax.axis_index("c")
    qw = wid * QPW
    pltpu.sync_copy(idx_hbm.at[pl.ds(qw * 3, QPW * 3)], idx_v)

    rows = [rows_a, rows_b]
    ws = [w_a, w_b]

    def fire(ch, buf):
        base = ch * QC * 3
        pltpu.async_copy(table_hbm.at[idx_v.at[pl.ds(base, 96)]],
                         rows[buf].at[pl.ds(0, 96)], sem)
        pltpu.async_copy(table_hbm.at[idx_v.at[pl.ds(base + 96, 96)]],
                         rows[buf].at[pl.ds(96, 96)], sem)
        pltpu.async_copy(w_hbm.at[pl.ds((qw * 3 + base) * 16, QC * 48)],
                         ws[buf], wsem)

    def drain(buf):
        pltpu.make_async_copy(table_hbm.at[idx_v.at[pl.ds(0, 96)]],
                              rows[buf].at[pl.ds(0, 96)], sem).wait()
        pltpu.make_async_copy(table_hbm.at[idx_v.at[pl.ds(0, 96)]],
                              rows[buf].at[pl.ds(96, 96)], sem).wait()
        pltpu.make_async_copy(w_hbm.at[pl.ds(0, QC * 48)], ws[buf],
                              wsem).wait()

    fire(0, 0)
    for ch in range(NCHUNK):
        buf = ch % 2
        if ch + 1 < NCHUNK:
            fire(ch + 1, 1 - buf)
        drain(buf)
        rows_v = rows[buf]
        w_v = ws[buf]

        def q_body(q, carry2):
            w0 = w_v[pl.ds((3 * q + 0) * 16, 16)]
            w1 = w_v[pl.ds((3 * q + 1) * 16, 16)]
            w2 = w_v[pl.ds((3 * q + 2) * 16, 16)]
            for c in range(C2 // 16):
                sl = pl.ds(c * 16, 16)
                acc = (rows_v[3 * q + 0, sl] * w0
                       + rows_v[3 * q + 1, sl] * w1
                       + rows_v[3 * q + 2, sl] * w2)
                out_v[q, sl] = acc
            return carry2

        lax.fori_loop(0, QC, q_body, 0)
        pltpu.sync_copy(out_v, out_hbm.at[pl.ds(qw + ch * QC, QC)])


def _sc_interp(idx_flat, w_flat, table):
    mesh = plsc.VectorSubcoreMesh(core_axis_name="c", subcore_axis_name="s")
    f = functools.partial(
        pl.kernel,
        out_type=jax.ShapeDtypeStruct((N1, C2), jnp.float32),
        mesh=mesh,
        scratch_types=[
            pltpu.VMEM((QPW * 3,), jnp.int32),
            pltpu.VMEM((QC * 3, C2), jnp.float32),
            pltpu.VMEM((QC * 3, C2), jnp.float32),
            pltpu.VMEM((QC * 48,), jnp.float32),
            pltpu.VMEM((QC * 48,), jnp.float32),
            pltpu.VMEM((QC, C2), jnp.float32),
            pltpu.SemaphoreType.DMA,
            pltpu.SemaphoreType.DMA,
        ],
    )(_sc_interp_body)
    return f(idx_flat, w_flat, table)


MLP_BLK = 1024
NB = N1 // MLP_BLK


def _mlp_body(p1_ref, it_ref, w1a_ref, w1b_ref, b1_ref, g1_ref, be1_ref,
              w2_ref, b2_ref, g2_ref, be2_ref, out_ref,
              y1s, y2s, st1, st2):
    p = pl.program_id(0)
    i = pl.program_id(1)
    rows = pl.ds(i * MLP_BLK, MLP_BLK)

    @pl.when(p == 0)
    def _():
        y = (jnp.dot(p1_ref[...], w1a_ref[...],
                     preferred_element_type=jnp.float32)
             + jnp.dot(it_ref[...], w1b_ref[...],
                       preferred_element_type=jnp.float32)
             + b1_ref[...])
        y1s[rows, :] = y
        st = jnp.concatenate([jnp.sum(y, axis=0, keepdims=True),
                              jnp.sum(y * y, axis=0, keepdims=True)], axis=0)

        @pl.when(i == 0)
        def _():
            st1[...] = st

        @pl.when(i != 0)
        def _():
            st1[...] += st

    @pl.when(p == 1)
    def _():
        mu = st1[0:1, :] * (1.0 / N1)
        var = st1[1:2, :] * (1.0 / N1) - mu * mu
        a = g1_ref[...] / jnp.sqrt(var + 1e-5)
        c = be1_ref[...] - mu * a
        x = jnp.maximum(y1s[rows, :] * a + c, 0.0)
        y = (jnp.dot(x, w2_ref[...], preferred_element_type=jnp.float32)
             + b2_ref[...])
        y2s[rows, :] = y
        st = jnp.concatenate([jnp.sum(y, axis=0, keepdims=True),
                              jnp.sum(y * y, axis=0, keepdims=True)], axis=0)

        @pl.when(i == 0)
        def _():
            st2[...] = st

        @pl.when(i != 0)
        def _():
            st2[...] += st

    @pl.when(p == 2)
    def _():
        mu = st2[0:1, :] * (1.0 / N1)
        var = st2[1:2, :] * (1.0 / N1) - mu * mu
        a = g2_ref[...] / jnp.sqrt(var + 1e-5)
        c = be2_ref[...] - mu * a
        out_ref[...] = jnp.maximum(y2s[rows, :] * a + c, 0.0)


def _mlp(points1, interp, W1, b1, g1, be1, W2, b2, g2, be2):
    w1a = W1[:C1]
    w1b = W1[C1:]
    zero = lambda p, i: (0, 0)
    ph0 = lambda p, i: (jnp.where(p == 0, i, 0), 0)
    ph2 = lambda p, i: (jnp.where(p == 2, i, 0), 0)
    return pl.pallas_call(
        _mlp_body,
        grid=(3, NB),
        in_specs=[
            pl.BlockSpec((MLP_BLK, C1), ph0),
            pl.BlockSpec((MLP_BLK, C2), ph0),
            pl.BlockSpec((C1, H1), zero),
            pl.BlockSpec((C2, H1), zero),
            pl.BlockSpec((1, H1), zero),
            pl.BlockSpec((1, H1), zero),
            pl.BlockSpec((1, H1), zero),
            pl.BlockSpec((H1, H2), zero),
            pl.BlockSpec((1, H2), zero),
            pl.BlockSpec((1, H2), zero),
            pl.BlockSpec((1, H2), zero),
        ],
        out_specs=pl.BlockSpec((MLP_BLK, H2), ph2),
        out_shape=jax.ShapeDtypeStruct((N1, H2), jnp.float32),
        scratch_shapes=[
            pltpu.VMEM((N1, H1), jnp.float32),
            pltpu.VMEM((N1, H2), jnp.float32),
            pltpu.VMEM((2, H1), jnp.float32),
            pltpu.VMEM((2, H2), jnp.float32),
        ],
    )(points1, interp, w1a, w1b, b1.reshape(1, H1), g1.reshape(1, H1),
      be1.reshape(1, H1), W2, b2.reshape(1, H2), g2.reshape(1, H2),
      be2.reshape(1, H2))


def kernel(xyz1, points1, offset1, xyz2, points2, offset2,
           W1, b1, g1, be1, W2, b2, g2, be2):
    del offset1, offset2
    xyz1p = jnp.concatenate(
        [xyz1, jnp.zeros((N1, 5), jnp.float32)], axis=1)
    kt = jnp.concatenate(
        [xyz2.T, jnp.zeros((5, N2), jnp.float32)], axis=0)
    idx, wspl = _knn(xyz1p, kt)
    interp = _sc_interp(idx.reshape(N1 * 3), wspl.reshape(N1 * 48), points2)
    return _mlp(points1, interp, W1, b1, g1, be1, W2, b2, g2, be2)


# speedup vs baseline: 14.6036x; 1.0523x over previous
import functools

import jax
import jax.numpy as jnp
from jax import lax
from jax.experimental import pallas as pl
from jax.experimental.pallas import tpu as pltpu
from jax.experimental.pallas import tpu_sc as plsc

N1 = 16384
N2 = 4096
C1 = 128
C2 = 256
H1 = 256
H2 = 128

KNN_BLK = 512

NWORK = 32
QPW = N1 // NWORK
QC = 64
NCHUNK = QPW // QC


def _knn_body(q_ref, kt_ref, idx_ref, wspl_ref):
    q = q_ref[...]
    kt = kt_ref[...]
    qk = jnp.dot(q, kt, preferred_element_type=jnp.float32)
    q2 = jnp.sum(q * q, axis=1, keepdims=True)
    k2 = jnp.sum(kt * kt, axis=0, keepdims=True)
    d2 = q2 - 2.0 * qk + k2
    iota = lax.broadcasted_iota(jnp.int32, (KNN_BLK, N2), 1).astype(jnp.float32)
    big = jnp.float32(3e38)
    inf = jnp.float32(jnp.inf)
    vals = []
    idxs = []
    for r in range(3):
        m = jnp.min(d2, axis=1, keepdims=True)
        cand = jnp.where(d2 == m, iota, big)
        im = jnp.min(cand, axis=1, keepdims=True)
        vals.append(m)
        idxs.append(im)
        if r < 2:
            d2 = jnp.where(iota == im, inf, d2)
    dist = [jnp.sqrt(jnp.maximum(v, 0.0)) for v in vals]
    rec = [1.0 / (d + 1e-8) for d in dist]
    s = rec[0] + rec[1] + rec[2]
    w = [r_ / s for r_ in rec]
    idx_ref[...] = jnp.concatenate(
        [i.astype(jnp.int32) for i in idxs], axis=1)
    wspl_ref[...] = jnp.concatenate(
        [jnp.broadcast_to(wi, (KNN_BLK, 16)) for wi in w], axis=1)


def _knn(xyz1p, kt):
    n = xyz1p.shape[0]
    grid = n // KNN_BLK
    return pl.pallas_call(
        _knn_body,
        grid=(grid,),
        in_specs=[
            pl.BlockSpec((KNN_BLK, 8), lambda i: (i, 0)),
            pl.BlockSpec((8, N2), lambda i: (0, 0)),
        ],
        out_specs=[
            pl.BlockSpec((KNN_BLK, 3), lambda i: (i, 0)),
            pl.BlockSpec((KNN_BLK, 48), lambda i: (i, 0)),
        ],
        out_shape=[
            jax.ShapeDtypeStruct((n, 3), jnp.int32),
            jax.ShapeDtypeStruct((n, 48), jnp.float32),
        ],
    )(xyz1p, kt)


def _make_sc_interp_body(qpw, nchunk):
    def _sc_interp_body(idx_hbm, w_hbm, table_hbm, out_hbm,
                        idx_v, rows_a, rows_b, w_a, w_b, out_v, sem, wsem):
        wid = lax.axis_index("s") * 2 + lax.axis_index("c")
        qw = wid * qpw
        pltpu.sync_copy(idx_hbm.at[pl.ds(qw * 3, qpw * 3)], idx_v)

        rows = [rows_a, rows_b]
        ws = [w_a, w_b]

        def fire(ch, buf):
            base = ch * QC * 3
            pltpu.async_copy(table_hbm.at[idx_v.at[pl.ds(base, 96)]],
                             rows[buf].at[pl.ds(0, 96)], sem)
            pltpu.async_copy(table_hbm.at[idx_v.at[pl.ds(base + 96, 96)]],
                             rows[buf].at[pl.ds(96, 96)], sem)
            pltpu.async_copy(w_hbm.at[pl.ds((qw * 3 + base) * 16, QC * 48)],
                             ws[buf], wsem)

        def drain(buf):
            pltpu.make_async_copy(table_hbm.at[idx_v.at[pl.ds(0, 96)]],
                                  rows[buf].at[pl.ds(0, 96)], sem).wait()
            pltpu.make_async_copy(table_hbm.at[idx_v.at[pl.ds(0, 96)]],
                                  rows[buf].at[pl.ds(96, 96)], sem).wait()
            pltpu.make_async_copy(w_hbm.at[pl.ds(0, QC * 48)], ws[buf],
                                  wsem).wait()

        fire(0, 0)
        for ch in range(nchunk):
            buf = ch % 2
            if ch + 1 < nchunk:
                fire(ch + 1, 1 - buf)
            drain(buf)
            rows_v = rows[buf]
            w_v = ws[buf]

            def q_body(q, carry2):
                w0 = w_v[pl.ds((3 * q + 0) * 16, 16)]
                w1 = w_v[pl.ds((3 * q + 1) * 16, 16)]
                w2 = w_v[pl.ds((3 * q + 2) * 16, 16)]
                for c in range(C2 // 16):
                    sl = pl.ds(c * 16, 16)
                    acc = (rows_v[3 * q + 0, sl] * w0
                           + rows_v[3 * q + 1, sl] * w1
                           + rows_v[3 * q + 2, sl] * w2)
                    out_v[q, sl] = acc
                return carry2

            lax.fori_loop(0, QC, q_body, 0)
            pltpu.sync_copy(out_v, out_hbm.at[pl.ds(qw + ch * QC, QC)])

    return _sc_interp_body


def _sc_interp(idx_flat, w_flat, table):
    nq = idx_flat.shape[0] // 3
    qpw = nq // NWORK
    nchunk = qpw // QC
    mesh = plsc.VectorSubcoreMesh(core_axis_name="c", subcore_axis_name="s")
    f = functools.partial(
        pl.kernel,
        out_type=jax.ShapeDtypeStruct((nq, C2), jnp.float32),
        mesh=mesh,
        scratch_types=[
            pltpu.VMEM((qpw * 3,), jnp.int32),
            pltpu.VMEM((QC * 3, C2), jnp.float32),
            pltpu.VMEM((QC * 3, C2), jnp.float32),
            pltpu.VMEM((QC * 48,), jnp.float32),
            pltpu.VMEM((QC * 48,), jnp.float32),
            pltpu.VMEM((QC, C2), jnp.float32),
            pltpu.SemaphoreType.DMA,
            pltpu.SemaphoreType.DMA,
        ],
    )(_make_sc_interp_body(qpw, nchunk))
    return f(idx_flat, w_flat, table)


MLP_BLK = 1024
NB = N1 // MLP_BLK


def _mlp_body(p1_ref, it_ref, w1a_ref, w1b_ref, b1_ref, g1_ref, be1_ref,
              w2_ref, b2_ref, g2_ref, be2_ref, out_ref,
              y1s, y2s, st1, st2):
    p = pl.program_id(0)
    i = pl.program_id(1)
    rows = pl.ds(i * MLP_BLK, MLP_BLK)

    @pl.when(p == 0)
    def _():
        y = (jnp.dot(p1_ref[...], w1a_ref[...],
                     preferred_element_type=jnp.float32)
             + jnp.dot(it_ref[...], w1b_ref[...],
                       preferred_element_type=jnp.float32)
             + b1_ref[...])
        y1s[rows, :] = y
        st = jnp.concatenate([jnp.sum(y, axis=0, keepdims=True),
                              jnp.sum(y * y, axis=0, keepdims=True)], axis=0)

        @pl.when(i == 0)
        def _():
            st1[...] = st

        @pl.when(i != 0)
        def _():
            st1[...] += st

    @pl.when(p == 1)
    def _():
        mu = st1[0:1, :] * (1.0 / N1)
        var = st1[1:2, :] * (1.0 / N1) - mu * mu
        a = g1_ref[...] / jnp.sqrt(var + 1e-5)
        c = be1_ref[...] - mu * a
        x = jnp.maximum(y1s[rows, :] * a + c, 0.0)
        y = (jnp.dot(x, w2_ref[...], preferred_element_type=jnp.float32)
             + b2_ref[...])
        y2s[rows, :] = y
        st = jnp.concatenate([jnp.sum(y, axis=0, keepdims=True),
                              jnp.sum(y * y, axis=0, keepdims=True)], axis=0)

        @pl.when(i == 0)
        def _():
            st2[...] = st

        @pl.when(i != 0)
        def _():
            st2[...] += st

    @pl.when(p == 2)
    def _():
        mu = st2[0:1, :] * (1.0 / N1)
        var = st2[1:2, :] * (1.0 / N1) - mu * mu
        a = g2_ref[...] / jnp.sqrt(var + 1e-5)
        c = be2_ref[...] - mu * a
        out_ref[...] = jnp.maximum(y2s[rows, :] * a + c, 0.0)


def _mlp(points1, interp, W1, b1, g1, be1, W2, b2, g2, be2):
    w1a = W1[:C1]
    w1b = W1[C1:]
    zero = lambda p, i: (0, 0)
    ph0 = lambda p, i: (jnp.where(p == 0, i, 0), 0)
    ph2 = lambda p, i: (jnp.where(p == 2, i, 0), 0)
    return pl.pallas_call(
        _mlp_body,
        grid=(3, NB),
        in_specs=[
            pl.BlockSpec((MLP_BLK, C1), ph0),
            pl.BlockSpec((MLP_BLK, C2), ph0),
            pl.BlockSpec((C1, H1), zero),
            pl.BlockSpec((C2, H1), zero),
            pl.BlockSpec((1, H1), zero),
            pl.BlockSpec((1, H1), zero),
            pl.BlockSpec((1, H1), zero),
            pl.BlockSpec((H1, H2), zero),
            pl.BlockSpec((1, H2), zero),
            pl.BlockSpec((1, H2), zero),
            pl.BlockSpec((1, H2), zero),
        ],
        out_specs=pl.BlockSpec((MLP_BLK, H2), ph2),
        out_shape=jax.ShapeDtypeStruct((N1, H2), jnp.float32),
        scratch_shapes=[
            pltpu.VMEM((N1, H1), jnp.float32),
            pltpu.VMEM((N1, H2), jnp.float32),
            pltpu.VMEM((2, H1), jnp.float32),
            pltpu.VMEM((2, H2), jnp.float32),
        ],
    )(points1, interp, w1a, w1b, b1.reshape(1, H1), g1.reshape(1, H1),
      be1.reshape(1, H1), W2, b2.reshape(1, H2), g2.reshape(1, H2),
      be2.reshape(1, H2))


def kernel(xyz1, points1, offset1, xyz2, points2, offset2,
           W1, b1, g1, be1, W2, b2, g2, be2):
    del offset1, offset2
    xyz1p = jnp.concatenate(
        [xyz1, jnp.zeros((N1, 5), jnp.float32)], axis=1)
    kt = jnp.concatenate(
        [xyz2.T, jnp.zeros((5, N2), jnp.float32)], axis=0)
    h = N1 // 2
    idx0, wspl0 = _knn(xyz1p[:h], kt)
    interp0 = _sc_interp(idx0.reshape(h * 3), wspl0.reshape(h * 48), points2)
    idx1, wspl1 = _knn(xyz1p[h:], kt)
    interp1 = _sc_interp(idx1.reshape(h * 3), wspl1.reshape(h * 48), points2)
    interp = jnp.concatenate([interp0, interp1], axis=0)
    return _mlp(points1, interp, W1, b1, g1, be1, W2, b2, g2, be2)


# speedup vs baseline: 15.2374x; 1.0434x over previous
import functools

import jax
import jax.numpy as jnp
from jax import lax
from jax.experimental import pallas as pl
from jax.experimental.pallas import tpu as pltpu
from jax.experimental.pallas import tpu_sc as plsc

N1 = 16384
N2 = 4096
C1 = 128
C2 = 256
H1 = 256
H2 = 128

KNN_BLK = 512

NWORK = 32
QPW = N1 // NWORK
QC = 64
NCHUNK = QPW // QC


def _knn_body(q_ref, kt_ref, idx_ref, wspl_ref):
    q = q_ref[...]
    kt = kt_ref[...]
    qk = jnp.dot(q, kt, preferred_element_type=jnp.float32)
    q2 = jnp.sum(q * q, axis=1, keepdims=True)
    k2 = jnp.sum(kt * kt, axis=0, keepdims=True)
    d2 = q2 - 2.0 * qk + k2
    iota = lax.broadcasted_iota(jnp.int32, (KNN_BLK, N2), 1).astype(jnp.float32)
    big = jnp.float32(3e38)
    inf = jnp.float32(jnp.inf)
    vals = []
    idxs = []
    for r in range(3):
        m = jnp.min(d2, axis=1, keepdims=True)
        cand = jnp.where(d2 == m, iota, big)
        im = jnp.min(cand, axis=1, keepdims=True)
        vals.append(m)
        idxs.append(im)
        if r < 2:
            d2 = jnp.where(iota == im, inf, d2)
    dist = [jnp.sqrt(jnp.maximum(v, 0.0)) for v in vals]
    rec = [1.0 / (d + 1e-8) for d in dist]
    s = rec[0] + rec[1] + rec[2]
    w = [r_ / s for r_ in rec]
    idx_ref[...] = jnp.concatenate(
        [i.astype(jnp.int32) for i in idxs], axis=1)
    wspl_ref[...] = jnp.concatenate(
        [jnp.broadcast_to(wi, (KNN_BLK, 16)) for wi in w], axis=1)


def _knn(xyz1p, kt):
    n = xyz1p.shape[0]
    grid = n // KNN_BLK
    return pl.pallas_call(
        _knn_body,
        grid=(grid,),
        in_specs=[
            pl.BlockSpec((KNN_BLK, 8), lambda i: (i, 0)),
            pl.BlockSpec((8, N2), lambda i: (0, 0)),
        ],
        out_specs=[
            pl.BlockSpec((KNN_BLK, 3), lambda i: (i, 0)),
            pl.BlockSpec((KNN_BLK, 48), lambda i: (i, 0)),
        ],
        out_shape=[
            jax.ShapeDtypeStruct((n, 3), jnp.int32),
            jax.ShapeDtypeStruct((n, 48), jnp.float32),
        ],
    )(xyz1p, kt)


def _make_sc_interp_body(qpw, nchunk):
    def _sc_interp_body(idx_hbm, w_hbm, table_hbm, out_hbm,
                        idx_v, rows_a, rows_b, w_a, w_b, out_v, sem, wsem):
        wid = lax.axis_index("s") * 2 + lax.axis_index("c")
        qw = wid * qpw
        pltpu.sync_copy(idx_hbm.at[pl.ds(qw * 3, qpw * 3)], idx_v)

        rows = [rows_a, rows_b]
        ws = [w_a, w_b]

        def fire(ch, buf):
            base = ch * QC * 3
            pltpu.async_copy(table_hbm.at[idx_v.at[pl.ds(base, 96)]],
                             rows[buf].at[pl.ds(0, 96)], sem)
            pltpu.async_copy(table_hbm.at[idx_v.at[pl.ds(base + 96, 96)]],
                             rows[buf].at[pl.ds(96, 96)], sem)
            pltpu.async_copy(w_hbm.at[pl.ds((qw * 3 + base) * 16, QC * 48)],
                             ws[buf], wsem)

        def drain(buf):
            pltpu.make_async_copy(table_hbm.at[idx_v.at[pl.ds(0, 96)]],
                                  rows[buf].at[pl.ds(0, 96)], sem).wait()
            pltpu.make_async_copy(table_hbm.at[idx_v.at[pl.ds(0, 96)]],
                                  rows[buf].at[pl.ds(96, 96)], sem).wait()
            pltpu.make_async_copy(w_hbm.at[pl.ds(0, QC * 48)], ws[buf],
                                  wsem).wait()

        fire(0, 0)
        for ch in range(nchunk):
            buf = ch % 2
            if ch + 1 < nchunk:
                fire(ch + 1, 1 - buf)
            drain(buf)
            rows_v = rows[buf]
            w_v = ws[buf]

            def q_body(q, carry2):
                w0 = w_v[pl.ds((3 * q + 0) * 16, 16)]
                w1 = w_v[pl.ds((3 * q + 1) * 16, 16)]
                w2 = w_v[pl.ds((3 * q + 2) * 16, 16)]
                for c in range(C2 // 16):
                    sl = pl.ds(c * 16, 16)
                    acc = (rows_v[3 * q + 0, sl] * w0
                           + rows_v[3 * q + 1, sl] * w1
                           + rows_v[3 * q + 2, sl] * w2)
                    out_v[q, sl] = acc
                return carry2

            lax.fori_loop(0, QC, q_body, 0)
            pltpu.sync_copy(out_v, out_hbm.at[pl.ds(qw + ch * QC, QC)])

    return _sc_interp_body


def _sc_interp(idx_flat, w_flat, table):
    nq = idx_flat.shape[0] // 3
    qpw = nq // NWORK
    nchunk = qpw // QC
    mesh = plsc.VectorSubcoreMesh(core_axis_name="c", subcore_axis_name="s")
    f = functools.partial(
        pl.kernel,
        out_type=jax.ShapeDtypeStruct((nq, C2), jnp.float32),
        mesh=mesh,
        scratch_types=[
            pltpu.VMEM((qpw * 3,), jnp.int32),
            pltpu.VMEM((QC * 3, C2), jnp.float32),
            pltpu.VMEM((QC * 3, C2), jnp.float32),
            pltpu.VMEM((QC * 48,), jnp.float32),
            pltpu.VMEM((QC * 48,), jnp.float32),
            pltpu.VMEM((QC, C2), jnp.float32),
            pltpu.SemaphoreType.DMA,
            pltpu.SemaphoreType.DMA,
        ],
    )(_make_sc_interp_body(qpw, nchunk))
    return f(idx_flat, w_flat, table)


MLP_BLK = 1024
NB = N1 // MLP_BLK


def _mlp_body(p1_ref, it0_ref, it1_ref, w1a_ref, w1b_ref, b1_ref, g1_ref,
              be1_ref, w2_ref, b2_ref, g2_ref, be2_ref, out_ref,
              y1s, y2s, st1, st2):
    p = pl.program_id(0)
    i = pl.program_id(1)
    rows = pl.ds(i * MLP_BLK, MLP_BLK)

    @pl.when(p == 0)
    def _():
        it = jnp.where(i < NB // 2, it0_ref[...], it1_ref[...])
        y = (jnp.dot(p1_ref[...], w1a_ref[...],
                     preferred_element_type=jnp.float32)
             + jnp.dot(it, w1b_ref[...],
                       preferred_element_type=jnp.float32)
             + b1_ref[...])
        y1s[rows, :] = y
        st = jnp.concatenate([jnp.sum(y, axis=0, keepdims=True),
                              jnp.sum(y * y, axis=0, keepdims=True)], axis=0)

        @pl.when(i == 0)
        def _():
            st1[...] = st

        @pl.when(i != 0)
        def _():
            st1[...] += st

    @pl.when(p == 1)
    def _():
        mu = st1[0:1, :] * (1.0 / N1)
        var = st1[1:2, :] * (1.0 / N1) - mu * mu
        a = g1_ref[...] / jnp.sqrt(var + 1e-5)
        c = be1_ref[...] - mu * a
        x = jnp.maximum(y1s[rows, :] * a + c, 0.0)
        y = (jnp.dot(x, w2_ref[...], preferred_element_type=jnp.float32)
             + b2_ref[...])
        y2s[rows, :] = y
        st = jnp.concatenate([jnp.sum(y, axis=0, keepdims=True),
                              jnp.sum(y * y, axis=0, keepdims=True)], axis=0)

        @pl.when(i == 0)
        def _():
            st2[...] = st

        @pl.when(i != 0)
        def _():
            st2[...] += st

    @pl.when(p == 2)
    def _():
        mu = st2[0:1, :] * (1.0 / N1)
        var = st2[1:2, :] * (1.0 / N1) - mu * mu
        a = g2_ref[...] / jnp.sqrt(var + 1e-5)
        c = be2_ref[...] - mu * a
        out_ref[...] = jnp.maximum(y2s[rows, :] * a + c, 0.0)


def _mlp(points1, interp0, interp1, W1, b1, g1, be1, W2, b2, g2, be2):
    w1a = W1[:C1]
    w1b = W1[C1:]
    nh = NB // 2
    zero = lambda p, i: (0, 0)
    ph0 = lambda p, i: (jnp.where(p == 0, i, 0), 0)
    ph0a = lambda p, i: (jnp.where((p == 0) & (i < nh), i, 0), 0)
    ph0b = lambda p, i: (jnp.where((p == 0) & (i >= nh), i - nh, 0), 0)
    ph2 = lambda p, i: (jnp.where(p == 2, i, 0), 0)
    return pl.pallas_call(
        _mlp_body,
        grid=(3, NB),
        in_specs=[
            pl.BlockSpec((MLP_BLK, C1), ph0),
            pl.BlockSpec((MLP_BLK, C2), ph0a),
            pl.BlockSpec((MLP_BLK, C2), ph0b),
            pl.BlockSpec((C1, H1), zero),
            pl.BlockSpec((C2, H1), zero),
            pl.BlockSpec((1, H1), zero),
            pl.BlockSpec((1, H1), zero),
            pl.BlockSpec((1, H1), zero),
            pl.BlockSpec((H1, H2), zero),
            pl.BlockSpec((1, H2), zero),
            pl.BlockSpec((1, H2), zero),
            pl.BlockSpec((1, H2), zero),
        ],
        out_specs=pl.BlockSpec((MLP_BLK, H2), ph2),
        out_shape=jax.ShapeDtypeStruct((N1, H2), jnp.float32),
        scratch_shapes=[
            pltpu.VMEM((N1, H1), jnp.float32),
            pltpu.VMEM((N1, H2), jnp.float32),
            pltpu.VMEM((2, H1), jnp.float32),
            pltpu.VMEM((2, H2), jnp.float32),
        ],
    )(points1, interp0, interp1, w1a, w1b, b1.reshape(1, H1),
      g1.reshape(1, H1), be1.reshape(1, H1), W2, b2.reshape(1, H2),
      g2.reshape(1, H2), be2.reshape(1, H2))


def kernel(xyz1, points1, offset1, xyz2, points2, offset2,
           W1, b1, g1, be1, W2, b2, g2, be2):
    del offset1, offset2
    xyz1p = jnp.concatenate(
        [xyz1, jnp.zeros((N1, 5), jnp.float32)], axis=1)
    kt = jnp.concatenate(
        [xyz2.T, jnp.zeros((5, N2), jnp.float32)], axis=0)
    h = N1 // 2
    idx0, wspl0 = _knn(xyz1p[:h], kt)
    interp0 = _sc_interp(idx0.reshape(h * 3), wspl0.reshape(h * 48), points2)
    idx1, wspl1 = _knn(xyz1p[h:], kt)
    interp1 = _sc_interp(idx1.reshape(h * 3), wspl1.reshape(h * 48), points2)
    return _mlp(points1, interp0, interp1, W1, b1, g1, be1, W2, b2, g2, be2)


# speedup vs baseline: 15.4182x; 1.0119x over previous
import functools

import jax
import jax.numpy as jnp
from jax import lax
from jax.experimental import pallas as pl
from jax.experimental.pallas import tpu as pltpu
from jax.experimental.pallas import tpu_sc as plsc

N1 = 16384
N2 = 4096
C1 = 128
C2 = 256
H1 = 256
H2 = 128

KNN_BLK = 512

NWORK = 32
QPW = N1 // NWORK
QC = 64
NCHUNK = QPW // QC


def _knn_body(q_ref, kt_ref, idx_ref, wspl_ref):
    q = q_ref[...]
    kt = kt_ref[...]
    qk = jnp.dot(q, kt, preferred_element_type=jnp.float32)
    q2 = jnp.sum(q * q, axis=1, keepdims=True)
    k2 = jnp.sum(kt * kt, axis=0, keepdims=True)
    d2 = k2 - 2.0 * qk
    iota = lax.broadcasted_iota(jnp.int32, (KNN_BLK, N2), 1).astype(jnp.float32)
    big = jnp.float32(3e38)
    inf = jnp.float32(jnp.inf)
    vals = []
    idxs = []
    for r in range(3):
        m = jnp.min(d2, axis=1, keepdims=True)
        eq = d2 == m
        im = jnp.min(jnp.where(eq, iota, big), axis=1, keepdims=True)
        vals.append(m)
        idxs.append(im)
        if r < 2:
            d2 = jnp.where(eq, inf, d2)
    dist = [jnp.sqrt(jnp.maximum(v + q2, 0.0)) for v in vals]
    rec = [1.0 / (d + 1e-8) for d in dist]
    s = rec[0] + rec[1] + rec[2]
    w = [r_ / s for r_ in rec]
    idx_ref[...] = jnp.concatenate(
        [i.astype(jnp.int32) for i in idxs], axis=1)
    wspl_ref[...] = jnp.concatenate(
        [jnp.broadcast_to(wi, (KNN_BLK, 16)) for wi in w], axis=1)


def _knn(xyz1p, kt):
    n = xyz1p.shape[0]
    grid = n // KNN_BLK
    return pl.pallas_call(
        _knn_body,
        grid=(grid,),
        in_specs=[
            pl.BlockSpec((KNN_BLK, 8), lambda i: (i, 0)),
            pl.BlockSpec((8, N2), lambda i: (0, 0)),
        ],
        out_specs=[
            pl.BlockSpec((KNN_BLK, 3), lambda i: (i, 0)),
            pl.BlockSpec((KNN_BLK, 48), lambda i: (i, 0)),
        ],
        out_shape=[
            jax.ShapeDtypeStruct((n, 3), jnp.int32),
            jax.ShapeDtypeStruct((n, 48), jnp.float32),
        ],
    )(xyz1p, kt)


def _make_sc_interp_body(qpw, nchunk):
    def _sc_interp_body(idx_hbm, w_hbm, table_hbm, out_hbm,
                        idx_v, rows_a, rows_b, w_a, w_b, out_v, sem, wsem):
        wid = lax.axis_index("s") * 2 + lax.axis_index("c")
        qw = wid * qpw
        pltpu.sync_copy(idx_hbm.at[pl.ds(qw * 3, qpw * 3)], idx_v)

        rows = [rows_a, rows_b]
        ws = [w_a, w_b]

        def fire(ch, buf):
            base = ch * QC * 3
            pltpu.async_copy(table_hbm.at[idx_v.at[pl.ds(base, 96)]],
                             rows[buf].at[pl.ds(0, 96)], sem)
            pltpu.async_copy(table_hbm.at[idx_v.at[pl.ds(base + 96, 96)]],
                             rows[buf].at[pl.ds(96, 96)], sem)
            pltpu.async_copy(w_hbm.at[pl.ds((qw * 3 + base) * 16, QC * 48)],
                             ws[buf], wsem)

        def drain(buf):
            pltpu.make_async_copy(table_hbm.at[idx_v.at[pl.ds(0, 96)]],
                                  rows[buf].at[pl.ds(0, 96)], sem).wait()
            pltpu.make_async_copy(table_hbm.at[idx_v.at[pl.ds(0, 96)]],
                                  rows[buf].at[pl.ds(96, 96)], sem).wait()
            pltpu.make_async_copy(w_hbm.at[pl.ds(0, QC * 48)], ws[buf],
                                  wsem).wait()

        fire(0, 0)
        for ch in range(nchunk):
            buf = ch % 2
            if ch + 1 < nchunk:
                fire(ch + 1, 1 - buf)
            drain(buf)
            rows_v = rows[buf]
            w_v = ws[buf]

            def q_body(q, carry2):
                w0 = w_v[pl.ds((3 * q + 0) * 16, 16)]
                w1 = w_v[pl.ds((3 * q + 1) * 16, 16)]
                w2 = w_v[pl.ds((3 * q + 2) * 16, 16)]
                for c in range(C2 // 16):
                    sl = pl.ds(c * 16, 16)
                    acc = (rows_v[3 * q + 0, sl] * w0
                           + rows_v[3 * q + 1, sl] * w1
                           + rows_v[3 * q + 2, sl] * w2)
                    out_v[q, sl] = acc
                return carry2

            lax.fori_loop(0, QC, q_body, 0)
            pltpu.sync_copy(out_v, out_hbm.at[pl.ds(qw + ch * QC, QC)])

    return _sc_interp_body


def _sc_interp(idx_flat, w_flat, table):
    nq = idx_flat.shape[0] // 3
    qpw = nq // NWORK
    nchunk = qpw // QC
    mesh = plsc.VectorSubcoreMesh(core_axis_name="c", subcore_axis_name="s")
    f = functools.partial(
        pl.kernel,
        out_type=jax.ShapeDtypeStruct((nq, C2), jnp.float32),
        mesh=mesh,
        scratch_types=[
            pltpu.VMEM((qpw * 3,), jnp.int32),
            pltpu.VMEM((QC * 3, C2), jnp.float32),
            pltpu.VMEM((QC * 3, C2), jnp.float32),
            pltpu.VMEM((QC * 48,), jnp.float32),
            pltpu.VMEM((QC * 48,), jnp.float32),
            pltpu.VMEM((QC, C2), jnp.float32),
            pltpu.SemaphoreType.DMA,
            pltpu.SemaphoreType.DMA,
        ],
    )(_make_sc_interp_body(qpw, nchunk))
    return f(idx_flat, w_flat, table)


MLP_BLK = 1024
NB = N1 // MLP_BLK


def _mlp_body(p1_ref, it0_ref, it1_ref, w1a_ref, w1b_ref, b1_ref, g1_ref,
              be1_ref, w2_ref, b2_ref, g2_ref, be2_ref, out_ref,
              y1s, y2s, st1, st2):
    p = pl.program_id(0)
    i = pl.program_id(1)
    rows = pl.ds(i * MLP_BLK, MLP_BLK)

    @pl.when(p == 0)
    def _():
        it = jnp.where(i < NB // 2, it0_ref[...], it1_ref[...])
        y = (jnp.dot(p1_ref[...], w1a_ref[...],
                     preferred_element_type=jnp.float32)
             + jnp.dot(it, w1b_ref[...],
                       preferred_element_type=jnp.float32)
             + b1_ref[...])
        y1s[rows, :] = y
        st = jnp.concatenate([jnp.sum(y, axis=0, keepdims=True),
                              jnp.sum(y * y, axis=0, keepdims=True)], axis=0)

        @pl.when(i == 0)
        def _():
            st1[...] = st

        @pl.when(i != 0)
        def _():
            st1[...] += st

    @pl.when(p == 1)
    def _():
        mu = st1[0:1, :] * (1.0 / N1)
        var = st1[1:2, :] * (1.0 / N1) - mu * mu
        a = g1_ref[...] / jnp.sqrt(var + 1e-5)
        c = be1_ref[...] - mu * a
        x = jnp.maximum(y1s[rows, :] * a + c, 0.0)
        y = (jnp.dot(x, w2_ref[...], preferred_element_type=jnp.float32)
             + b2_ref[...])
        y2s[rows, :] = y
        st = jnp.concatenate([jnp.sum(y, axis=0, keepdims=True),
                              jnp.sum(y * y, axis=0, keepdims=True)], axis=0)

        @pl.when(i == 0)
        def _():
            st2[...] = st

        @pl.when(i != 0)
        def _():
            st2[...] += st

    @pl.when(p == 2)
    def _():
        mu = st2[0:1, :] * (1.0 / N1)
        var = st2[1:2, :] * (1.0 / N1) - mu * mu
        a = g2_ref[...] / jnp.sqrt(var + 1e-5)
        c = be2_ref[...] - mu * a
        out_ref[...] = jnp.maximum(y2s[rows, :] * a + c, 0.0)


def _mlp(points1, interp0, interp1, W1, b1, g1, be1, W2, b2, g2, be2):
    w1a = W1[:C1]
    w1b = W1[C1:]
    nh = NB // 2
    zero = lambda p, i: (0, 0)
    ph0 = lambda p, i: (jnp.where(p == 0, i, 0), 0)
    ph0a = lambda p, i: (jnp.where((p == 0) & (i < nh), i, 0), 0)
    ph0b = lambda p, i: (jnp.where((p == 0) & (i >= nh), i - nh, 0), 0)
    ph2 = lambda p, i: (jnp.where(p == 2, i, 0), 0)
    return pl.pallas_call(
        _mlp_body,
        grid=(3, NB),
        in_specs=[
            pl.BlockSpec((MLP_BLK, C1), ph0),
            pl.BlockSpec((MLP_BLK, C2), ph0a),
            pl.BlockSpec((MLP_BLK, C2), ph0b),
            pl.BlockSpec((C1, H1), zero),
            pl.BlockSpec((C2, H1), zero),
            pl.BlockSpec((1, H1), zero),
            pl.BlockSpec((1, H1), zero),
            pl.BlockSpec((1, H1), zero),
            pl.BlockSpec((H1, H2), zero),
            pl.BlockSpec((1, H2), zero),
            pl.BlockSpec((1, H2), zero),
            pl.BlockSpec((1, H2), zero),
        ],
        out_specs=pl.BlockSpec((MLP_BLK, H2), ph2),
        out_shape=jax.ShapeDtypeStruct((N1, H2), jnp.float32),
        scratch_shapes=[
            pltpu.VMEM((N1, H1), jnp.float32),
            pltpu.VMEM((N1, H2), jnp.float32),
            pltpu.VMEM((2, H1), jnp.float32),
            pltpu.VMEM((2, H2), jnp.float32),
        ],
    )(points1, interp0, interp1, w1a, w1b, b1.reshape(1, H1),
      g1.reshape(1, H1), be1.reshape(1, H1), W2, b2.reshape(1, H2),
      g2.reshape(1, H2), be2.reshape(1, H2))


def kernel(xyz1, points1, offset1, xyz2, points2, offset2,
           W1, b1, g1, be1, W2, b2, g2, be2):
    del offset1, offset2
    xyz1p = jnp.concatenate(
        [xyz1, jnp.zeros((N1, 5), jnp.float32)], axis=1)
    kt = jnp.concatenate(
        [xyz2.T, jnp.zeros((5, N2), jnp.float32)], axis=0)
    h = N1 // 2
    idx0, wspl0 = _knn(xyz1p[:h], kt)
    interp0 = _sc_interp(idx0.reshape(h * 3), wspl0.reshape(h * 48), points2)
    idx1, wspl1 = _knn(xyz1p[h:], kt)
    interp1 = _sc_interp(idx1.reshape(h * 3), wspl1.reshape(h * 48), points2)
    return _mlp(points1, interp0, interp1, W1, b1, g1, be1, W2, b2, g2, be2)


# speedup vs baseline: 15.4441x; 1.0017x over previous
import functools

import jax
import jax.numpy as jnp
from jax import lax
from jax.experimental import pallas as pl
from jax.experimental.pallas import tpu as pltpu
from jax.experimental.pallas import tpu_sc as plsc

N1 = 16384
N2 = 4096
C1 = 128
C2 = 256
H1 = 256
H2 = 128

KNN_BLK = 512

NWORK = 32
QPW = N1 // NWORK
QC = 64
NCHUNK = QPW // QC


def _knn_body(q_ref, kt_ref, idx_ref, wspl_ref):
    q = q_ref[...]
    kt = kt_ref[...]
    qk = jnp.dot(q, kt, preferred_element_type=jnp.float32)
    q2 = jnp.sum(q * q, axis=1, keepdims=True)
    k2 = jnp.sum(kt * kt, axis=0, keepdims=True)
    d2 = q2 - 2.0 * qk + k2
    iota = lax.broadcasted_iota(jnp.int32, (KNN_BLK, N2), 1).astype(jnp.float32)
    big = jnp.float32(3e38)
    inf = jnp.float32(jnp.inf)
    vals = []
    idxs = []
    for r in range(3):
        m = jnp.min(d2, axis=1, keepdims=True)
        eq = d2 == m
        im = jnp.min(jnp.where(eq, iota, big), axis=1, keepdims=True)
        vals.append(m)
        idxs.append(im)
        if r < 2:
            d2 = jnp.where(eq, inf, d2)
    dist = [jnp.sqrt(jnp.maximum(v, 0.0)) for v in vals]
    rec = [1.0 / (d + 1e-8) for d in dist]
    s = rec[0] + rec[1] + rec[2]
    w = [r_ / s for r_ in rec]
    idx_ref[...] = jnp.concatenate(
        [i.astype(jnp.int32) for i in idxs], axis=1)
    wspl_ref[...] = jnp.concatenate(
        [jnp.broadcast_to(wi, (KNN_BLK, 16)) for wi in w], axis=1)


def _knn(xyz1p, kt):
    n = xyz1p.shape[0]
    grid = n // KNN_BLK
    return pl.pallas_call(
        _knn_body,
        grid=(grid,),
        in_specs=[
            pl.BlockSpec((KNN_BLK, 8), lambda i: (i, 0)),
            pl.BlockSpec((8, N2), lambda i: (0, 0)),
        ],
        out_specs=[
            pl.BlockSpec((KNN_BLK, 3), lambda i: (i, 0)),
            pl.BlockSpec((KNN_BLK, 48), lambda i: (i, 0)),
        ],
        out_shape=[
            jax.ShapeDtypeStruct((n, 3), jnp.int32),
            jax.ShapeDtypeStruct((n, 48), jnp.float32),
        ],
    )(xyz1p, kt)


def _make_sc_interp_body(qpw, nchunk):
    def _sc_interp_body(idx_hbm, w_hbm, table_hbm, out_hbm,
                        idx_v, rows_a, rows_b, w_a, w_b, out_v, sem, wsem):
        wid = lax.axis_index("s") * 2 + lax.axis_index("c")
        qw = wid * qpw
        pltpu.sync_copy(idx_hbm.at[pl.ds(qw * 3, qpw * 3)], idx_v)

        rows = [rows_a, rows_b]
        ws = [w_a, w_b]

        def fire(ch, buf):
            base = ch * QC * 3
            pltpu.async_copy(table_hbm.at[idx_v.at[pl.ds(base, 96)]],
                             rows[buf].at[pl.ds(0, 96)], sem)
            pltpu.async_copy(table_hbm.at[idx_v.at[pl.ds(base + 96, 96)]],
                             rows[buf].at[pl.ds(96, 96)], sem)
            pltpu.async_copy(w_hbm.at[pl.ds((qw * 3 + base) * 16, QC * 48)],
                             ws[buf], wsem)

        def drain(buf):
            pltpu.make_async_copy(table_hbm.at[idx_v.at[pl.ds(0, 96)]],
                                  rows[buf].at[pl.ds(0, 96)], sem).wait()
            pltpu.make_async_copy(table_hbm.at[idx_v.at[pl.ds(0, 96)]],
                                  rows[buf].at[pl.ds(96, 96)], sem).wait()
            pltpu.make_async_copy(w_hbm.at[pl.ds(0, QC * 48)], ws[buf],
                                  wsem).wait()

        fire(0, 0)
        for ch in range(nchunk):
            buf = ch % 2
            if ch + 1 < nchunk:
                fire(ch + 1, 1 - buf)
            drain(buf)
            rows_v = rows[buf]
            w_v = ws[buf]

            def q_body(q, carry2):
                w0 = w_v[pl.ds((3 * q + 0) * 16, 16)]
                w1 = w_v[pl.ds((3 * q + 1) * 16, 16)]
                w2 = w_v[pl.ds((3 * q + 2) * 16, 16)]
                for c in range(C2 // 16):
                    sl = pl.ds(c * 16, 16)
                    acc = (rows_v[3 * q + 0, sl] * w0
                           + rows_v[3 * q + 1, sl] * w1
                           + rows_v[3 * q + 2, sl] * w2)
                    out_v[q, sl] = acc
                return carry2

            lax.fori_loop(0, QC, q_body, 0)
            pltpu.sync_copy(out_v, out_hbm.at[pl.ds(qw + ch * QC, QC)])

    return _sc_interp_body


def _sc_interp(idx_flat, w_flat, table):
    nq = idx_flat.shape[0] // 3
    qpw = nq // NWORK
    nchunk = qpw // QC
    mesh = plsc.VectorSubcoreMesh(core_axis_name="c", subcore_axis_name="s")
    f = functools.partial(
        pl.kernel,
        out_type=jax.ShapeDtypeStruct((nq, C2), jnp.float32),
        mesh=mesh,
        scratch_types=[
            pltpu.VMEM((qpw * 3,), jnp.int32),
            pltpu.VMEM((QC * 3, C2), jnp.float32),
            pltpu.VMEM((QC * 3, C2), jnp.float32),
            pltpu.VMEM((QC * 48,), jnp.float32),
            pltpu.VMEM((QC * 48,), jnp.float32),
            pltpu.VMEM((QC, C2), jnp.float32),
            pltpu.SemaphoreType.DMA,
            pltpu.SemaphoreType.DMA,
        ],
    )(_make_sc_interp_body(qpw, nchunk))
    return f(idx_flat, w_flat, table)


MLP_BLK = 1024
NB = N1 // MLP_BLK


def _mlp_body(p1_ref, it0_ref, it1_ref, w1a_ref, w1b_ref, b1_ref, g1_ref,
              be1_ref, w2_ref, b2_ref, g2_ref, be2_ref, out_ref,
              y1s, y2s, st1, st2):
    p = pl.program_id(0)
    i = pl.program_id(1)
    rows = pl.ds(i * MLP_BLK, MLP_BLK)

    @pl.when(p == 0)
    def _():
        it = jnp.where(i < NB // 2, it0_ref[...], it1_ref[...])
        y = (jnp.dot(p1_ref[...], w1a_ref[...],
                     preferred_element_type=jnp.float32)
             + jnp.dot(it, w1b_ref[...],
                       preferred_element_type=jnp.float32)
             + b1_ref[...])
        y1s[rows, :] = y
        st = jnp.concatenate([jnp.sum(y, axis=0, keepdims=True),
                              jnp.sum(y * y, axis=0, keepdims=True)], axis=0)

        @pl.when(i == 0)
        def _():
            st1[...] = st

        @pl.when(i != 0)
        def _():
            st1[...] += st

    @pl.when(p == 1)
    def _():
        mu = st1[0:1, :] * (1.0 / N1)
        var = st1[1:2, :] * (1.0 / N1) - mu * mu
        a = g1_ref[...] / jnp.sqrt(var + 1e-5)
        c = be1_ref[...] - mu * a
        x = jnp.maximum(y1s[rows, :] * a + c, 0.0)
        y = (jnp.dot(x, w2_ref[...], preferred_element_type=jnp.float32)
             + b2_ref[...])
        y2s[rows, :] = y
        st = jnp.concatenate([jnp.sum(y, axis=0, keepdims=True),
                              jnp.sum(y * y, axis=0, keepdims=True)], axis=0)

        @pl.when(i == 0)
        def _():
            st2[...] = st

        @pl.when(i != 0)
        def _():
            st2[...] += st

    @pl.when(p == 2)
    def _():
        mu = st2[0:1, :] * (1.0 / N1)
        var = st2[1:2, :] * (1.0 / N1) - mu * mu
        a = g2_ref[...] / jnp.sqrt(var + 1e-5)
        c = be2_ref[...] - mu * a
        out_ref[...] = jnp.maximum(y2s[rows, :] * a + c, 0.0)


def _mlp(points1, interp0, interp1, W1, b1, g1, be1, W2, b2, g2, be2):
    w1a = W1[:C1]
    w1b = W1[C1:]
    nh = NB // 2
    zero = lambda p, i: (0, 0)
    ph0 = lambda p, i: (jnp.where(p == 0, i, 0), 0)
    ph0a = lambda p, i: (jnp.where((p == 0) & (i < nh), i, 0), 0)
    ph0b = lambda p, i: (jnp.where((p == 0) & (i >= nh), i - nh, 0), 0)
    ph2 = lambda p, i: (jnp.where(p == 2, i, 0), 0)
    return pl.pallas_call(
        _mlp_body,
        grid=(3, NB),
        in_specs=[
            pl.BlockSpec((MLP_BLK, C1), ph0),
            pl.BlockSpec((MLP_BLK, C2), ph0a),
            pl.BlockSpec((MLP_BLK, C2), ph0b),
            pl.BlockSpec((C1, H1), zero),
            pl.BlockSpec((C2, H1), zero),
            pl.BlockSpec((1, H1), zero),
            pl.BlockSpec((1, H1), zero),
            pl.BlockSpec((1, H1), zero),
            pl.BlockSpec((H1, H2), zero),
            pl.BlockSpec((1, H2), zero),
            pl.BlockSpec((1, H2), zero),
            pl.BlockSpec((1, H2), zero),
        ],
        out_specs=pl.BlockSpec((MLP_BLK, H2), ph2),
        out_shape=jax.ShapeDtypeStruct((N1, H2), jnp.float32),
        scratch_shapes=[
            pltpu.VMEM((N1, H1), jnp.float32),
            pltpu.VMEM((N1, H2), jnp.float32),
            pltpu.VMEM((2, H1), jnp.float32),
            pltpu.VMEM((2, H2), jnp.float32),
        ],
    )(points1, interp0, interp1, w1a, w1b, b1.reshape(1, H1),
      g1.reshape(1, H1), be1.reshape(1, H1), W2, b2.reshape(1, H2),
      g2.reshape(1, H2), be2.reshape(1, H2))


def kernel(xyz1, points1, offset1, xyz2, points2, offset2,
           W1, b1, g1, be1, W2, b2, g2, be2):
    del offset1, offset2
    xyz1p = jnp.concatenate(
        [xyz1, jnp.zeros((N1, 5), jnp.float32)], axis=1)
    kt = jnp.concatenate(
        [xyz2.T, jnp.zeros((5, N2), jnp.float32)], axis=0)
    h = N1 // 2
    idx0, wspl0 = _knn(xyz1p[:h], kt)
    interp0 = _sc_interp(idx0.reshape(h * 3), wspl0.reshape(h * 48), points2)
    idx1, wspl1 = _knn(xyz1p[h:], kt)
    interp1 = _sc_interp(idx1.reshape(h * 3), wspl1.reshape(h * 48), points2)
    return _mlp(points1, interp0, interp1, W1, b1, g1, be1, W2, b2, g2, be2)
